# baseline (speedup 1.0000x reference)
import functools

import jax
import jax.numpy as jnp
import numpy as np
from jax.experimental import pallas as pl
from jax.experimental.pallas import tpu as pltpu

_P_DROP = 0.1
_P_UNARY = 0.5

_CHUNK = 2000
_DEPTH = 16


def _rotl(x, r):
    return ((x << np.uint32(r)) | (x >> np.uint32(32 - r))).astype(np.uint32)


def _threefry2x32(k0, k1, x0, x1):
    rotations = ((13, 15, 26, 6), (17, 29, 16, 24))
    ks = (np.uint32(k0), np.uint32(k1),
          np.uint32(k0 ^ k1 ^ np.uint32(0x1BD11BDA)))
    x0 = (x0 + ks[0]).astype(np.uint32)
    x1 = (x1 + ks[1]).astype(np.uint32)
    for i in range(5):
        for r in rotations[i % 2]:
            x0 = (x0 + x1).astype(np.uint32)
            x1 = _rotl(x1, r)
            x1 = x1 ^ x0
        x0 = (x0 + ks[(i + 1) % 3]).astype(np.uint32)
        x1 = (x1 + ks[(i + 2) % 3] + np.uint32(i + 1)).astype(np.uint32)
    return x0, x1


def _bernoulli(key, p, n):
    o0, o1 = _threefry2x32(key[0], key[1], np.zeros(n, np.uint32),
                           np.arange(n, dtype=np.uint32))
    bits = o0 ^ o1
    fb = ((bits >> np.uint32(9)) | np.uint32(0x3F800000)).view(np.float32)
    u = np.maximum(np.float32(0.0), fb - np.float32(1.0))
    return u < np.float32(p)


@functools.lru_cache(maxsize=None)
def _row_drop(n: int):
    s0, s1 = _threefry2x32(np.uint32(0), np.uint32(42),
                           np.zeros(2, np.uint32),
                           np.arange(2, dtype=np.uint32))
    drop = _bernoulli((s0[0], s1[0]), _P_DROP, n)
    unary = _bernoulli((s0[1], s1[1]), _P_UNARY, n)
    return ((drop & (~unary)).astype(np.float32),
            (drop & unary).astype(np.float32))


def _body(chunk, depth, e_ref, g_ref, w_ref, m_hbm, x_hbm, o_hbm,
          inbuf, outbuf, mbuf, in_sem, out_sem, m_sem):
    n, d = x_hbm.shape
    nc = n // chunk

    def in_copy(i, slot):
        return pltpu.make_async_copy(
            x_hbm.at[pl.ds(i * chunk, chunk), :],
            inbuf.at[slot],
            in_sem.at[slot],
        )

    def m_copy(i, slot):
        return pltpu.make_async_copy(
            m_hbm.at[i],
            mbuf.at[slot],
            m_sem.at[slot],
        )

    def out_copy(i, slot):
        return pltpu.make_async_copy(
            outbuf.at[slot],
            o_hbm.at[pl.ds(i * chunk, chunk), :],
            out_sem.at[slot],
        )

    for i in range(depth):
        m_copy(i, i).start()
        in_copy(i, i).start()

    def step(i, carry):
        slot = jax.lax.rem(i, depth)
        in_copy(i, slot).wait()
        m_copy(i, slot).wait()

        @pl.when(i >= depth)
        def _():
            out_copy(i - depth, slot).wait()

        x = inbuf[slot]
        p = jax.lax.dot_general(e_ref[...], mbuf[slot],
                                (((1,), (0,)), ((), ())),
                                preferred_element_type=jnp.float32)
        c = (p.astype(jnp.bfloat16) * g_ref[...])
        t = jax.lax.dot_general(c, w_ref[...], (((1,), (0,)), ((), ())),
                                preferred_element_type=jnp.float32)
        outbuf[slot] = x - x * t
        out_copy(i, slot).start()

        @pl.when(i + depth < nc)
        def _():
            m_copy(i + depth, slot).start()
            in_copy(i + depth, slot).start()

        return carry

    jax.lax.fori_loop(0, nc, step, 0, unroll=False)

    for i in range(nc - depth, nc):
        out_copy(i, i % depth).wait()


@functools.lru_cache(maxsize=None)
def _mask_tiles(n: int, chunk: int):
    import ml_dtypes
    bf16 = ml_dtypes.bfloat16
    a, b = _row_drop(n)
    nc = n // chunk
    tr = chunk // 8
    ab = np.zeros((nc, 256, 128), dtype=bf16)
    ab[:, :tr, 0::16] = a.reshape(nc, tr, 8).astype(bf16)
    ab[:, :tr, 1::16] = b.reshape(nc, tr, 8).astype(bf16)

    e = np.zeros((chunk, 256), dtype=bf16)
    e[np.arange(chunk), np.arange(chunk) // 8] = 1.0

    g = np.zeros((chunk, 128), dtype=bf16)
    g[np.arange(chunk)[:, None], (np.arange(chunk) % 8)[:, None] * 16
      + np.arange(16)[None, :]] = 1.0

    w = np.zeros((128, 128), dtype=bf16)
    w[0::16, 0:3] = 1.0
    w[1::16, 127] = 1.0
    return ab, e, g, w


def kernel(data):
    n, d = data.shape
    chunk = next(c for c in (_CHUNK, 1000, 500, 8) if n % c == 0)
    depth = min(_DEPTH, n // chunk)

    ab, e, g, w = _mask_tiles(n, chunk)
    masks = jnp.asarray(ab)

    return pl.pallas_call(
        functools.partial(_body, chunk, depth),
        in_specs=[
            pl.BlockSpec(memory_space=pltpu.MemorySpace.VMEM),
            pl.BlockSpec(memory_space=pltpu.MemorySpace.VMEM),
            pl.BlockSpec(memory_space=pltpu.MemorySpace.VMEM),
            pl.BlockSpec(memory_space=pltpu.MemorySpace.HBM),
            pl.BlockSpec(memory_space=pltpu.MemorySpace.HBM),
        ],
        out_specs=pl.BlockSpec(memory_space=pltpu.MemorySpace.HBM),
        out_shape=jax.ShapeDtypeStruct((n, d), data.dtype),
        scratch_shapes=[
            pltpu.VMEM((depth, chunk, d), jnp.float32),
            pltpu.VMEM((depth, chunk, d), jnp.float32),
            pltpu.VMEM((depth, 256, 128), jnp.bfloat16),
            pltpu.SemaphoreType.DMA((depth,)),
            pltpu.SemaphoreType.DMA((depth,)),
            pltpu.SemaphoreType.DMA((depth,)),
        ],
    )(jnp.asarray(e), jnp.asarray(g), jnp.asarray(w), masks, data)


# speedup vs baseline: 1.2731x; 1.2731x over previous
import functools

import jax
import jax.numpy as jnp
import numpy as np
from jax.experimental import pallas as pl
from jax.experimental.pallas import tpu as pltpu

_P_DROP = 0.1
_P_UNARY = 0.5

_CHUNK = 2000
_DEPTH = 16


def _rotl(x, r):
    return ((x << np.uint32(r)) | (x >> np.uint32(32 - r))).astype(np.uint32)


def _threefry2x32(k0, k1, x0, x1):
    rotations = ((13, 15, 26, 6), (17, 29, 16, 24))
    ks = (np.uint32(k0), np.uint32(k1),
          np.uint32(k0 ^ k1 ^ np.uint32(0x1BD11BDA)))
    x0 = (x0 + ks[0]).astype(np.uint32)
    x1 = (x1 + ks[1]).astype(np.uint32)
    for i in range(5):
        for r in rotations[i % 2]:
            x0 = (x0 + x1).astype(np.uint32)
            x1 = _rotl(x1, r)
            x1 = x1 ^ x0
        x0 = (x0 + ks[(i + 1) % 3]).astype(np.uint32)
        x1 = (x1 + ks[(i + 2) % 3] + np.uint32(i + 1)).astype(np.uint32)
    return x0, x1


def _bernoulli(key, p, n):
    o0, o1 = _threefry2x32(key[0], key[1], np.zeros(n, np.uint32),
                           np.arange(n, dtype=np.uint32))
    bits = o0 ^ o1
    fb = ((bits >> np.uint32(9)) | np.uint32(0x3F800000)).view(np.float32)
    u = np.maximum(np.float32(0.0), fb - np.float32(1.0))
    return u < np.float32(p)


@functools.lru_cache(maxsize=None)
def _row_drop(n: int):
    s0, s1 = _threefry2x32(np.uint32(0), np.uint32(42),
                           np.zeros(2, np.uint32),
                           np.arange(2, dtype=np.uint32))
    drop = _bernoulli((s0[0], s1[0]), _P_DROP, n)
    unary = _bernoulli((s0[1], s1[1]), _P_UNARY, n)
    return ((drop & (~unary)).astype(np.float32),
            (drop & unary).astype(np.float32))


def _body(chunk, depth, e_ref, g_ref, w_ref, m_hbm, x_hbm, o_hbm,
          inbuf, outbuf, mbuf, in_sem, out_sem, m_sem):
    n, d = x_hbm.shape
    nc = n // chunk

    def in_copy(i, slot):
        return pltpu.make_async_copy(
            x_hbm.at[pl.ds(i * chunk, chunk), :],
            inbuf.at[slot],
            in_sem.at[slot],
        )

    def m_copy(i, slot):
        return pltpu.make_async_copy(
            m_hbm.at[i],
            mbuf.at[slot],
            m_sem.at[slot],
        )

    def out_copy(i, slot):
        return pltpu.make_async_copy(
            outbuf.at[slot],
            o_hbm.at[pl.ds(i * chunk, chunk), :],
            out_sem.at[slot],
        )

    for i in range(depth):
        m_copy(i, i).start()
        in_copy(i, i).start()

    def step(i, carry):
        slot = jax.lax.rem(i, depth)
        in_copy(i, slot).wait()
        m_copy(i, slot).wait()

        @pl.when(i >= depth)
        def _():
            out_copy(i - depth, slot).wait()

        x = inbuf[slot]
        ab = mbuf[slot][: chunk // 8]
        p = jnp.broadcast_to(ab[:, None, :], (chunk // 8, 8, 128))
        p = p.reshape(chunk, 128)
        c = p * g_ref[...]
        t = jax.lax.dot_general(c, w_ref[...], (((1,), (0,)), ((), ())),
                                preferred_element_type=jnp.float32)
        outbuf[slot] = x - x * t
        out_copy(i, slot).start()

        @pl.when(i + depth < nc)
        def _():
            m_copy(i + depth, slot).start()
            in_copy(i + depth, slot).start()

        return carry

    jax.lax.fori_loop(0, nc, step, 0, unroll=False)

    for i in range(nc - depth, nc):
        out_copy(i, i % depth).wait()


@functools.lru_cache(maxsize=None)
def _mask_tiles(n: int, chunk: int):
    import ml_dtypes
    bf16 = ml_dtypes.bfloat16
    a, b = _row_drop(n)
    nc = n // chunk
    tr = chunk // 8
    ab = np.zeros((nc, 256, 128), dtype=bf16)
    ab[:, :tr, 0::16] = a.reshape(nc, tr, 8).astype(bf16)
    ab[:, :tr, 1::16] = b.reshape(nc, tr, 8).astype(bf16)

    e = np.zeros((chunk, 256), dtype=bf16)
    e[np.arange(chunk), np.arange(chunk) // 8] = 1.0

    g = np.zeros((chunk, 128), dtype=bf16)
    g[np.arange(chunk)[:, None], (np.arange(chunk) % 8)[:, None] * 16
      + np.arange(16)[None, :]] = 1.0

    w = np.zeros((128, 128), dtype=bf16)
    w[0::16, 0:3] = 1.0
    w[1::16, 127] = 1.0
    return ab, e, g, w


def kernel(data):
    n, d = data.shape
    chunk = next(c for c in (_CHUNK, 1000, 500, 8) if n % c == 0)
    depth = min(_DEPTH, n // chunk)

    ab, e, g, w = _mask_tiles(n, chunk)
    masks = jnp.asarray(ab)

    return pl.pallas_call(
        functools.partial(_body, chunk, depth),
        in_specs=[
            pl.BlockSpec(memory_space=pltpu.MemorySpace.VMEM),
            pl.BlockSpec(memory_space=pltpu.MemorySpace.VMEM),
            pl.BlockSpec(memory_space=pltpu.MemorySpace.VMEM),
            pl.BlockSpec(memory_space=pltpu.MemorySpace.HBM),
            pl.BlockSpec(memory_space=pltpu.MemorySpace.HBM),
        ],
        out_specs=pl.BlockSpec(memory_space=pltpu.MemorySpace.HBM),
        out_shape=jax.ShapeDtypeStruct((n, d), data.dtype),
        scratch_shapes=[
            pltpu.VMEM((depth, chunk, d), jnp.float32),
            pltpu.VMEM((depth, chunk, d), jnp.float32),
            pltpu.VMEM((depth, 256, 128), jnp.bfloat16),
            pltpu.SemaphoreType.DMA((depth,)),
            pltpu.SemaphoreType.DMA((depth,)),
            pltpu.SemaphoreType.DMA((depth,)),
        ],
    )(jnp.asarray(e), jnp.asarray(g), jnp.asarray(w), masks, data)
